# Initial kernel scaffold; baseline (speedup 1.0000x reference)
import jax
import jax.numpy as jnp
from jax.experimental import pallas as pl


def kernel(edge_index, X, W, b):
    raise NotImplementedError("write your pallas kernel here")


# speedup vs baseline: 7.2270x; 7.2270x over previous
import jax
import jax.numpy as jnp
from jax import lax
from jax.experimental import pallas as pl
from jax.experimental.pallas import tpu as pltpu
from jax.experimental.pallas import tpu_sc as plsc

N_NODES = 50000
N_EDGES = 800000
N_FEAT = 128
N_EMB = 64
N_LAYERS = 3
ALPHA = 1.0 / (N_LAYERS + 1)

NC = 2
NS = 16
NW = NC * NS
CHUNK = 128
QW = 16

N_PAD = 50176
RPT = N_PAD // NS
BLK = 196
NBLK = RPT // BLK

PCH = 392
P_PAD = NS * PCH * CHUNK
JBLK = 49
NJB = PCH // JBLK
FCH = 196
F_PAD = NW * FCH * CHUNK

BN = 1024


def _mesh():
    return plsc.VectorSubcoreMesh(core_axis_name="c", subcore_axis_name="s")


_SC_PARAMS = pltpu.CompilerParams(use_tc_tiling_on_sc=False)


def _fill_rows(ref, n, value):
    val = jnp.full((QW,), value, jnp.float32)

    def row(i, carry):
        ref[i, :] = val
        return carry

    lax.fori_loop(0, n, row, 0)


def _rsqrt16(d):
    i = lax.bitcast_convert_type(d, jnp.int32)
    i = 0x5F3759DF - lax.shift_right_logical(i, 1)
    y = lax.bitcast_convert_type(i, jnp.float32)
    for _ in range(3):
        y = y * (1.5 - 0.5 * d * y * y)
    return y


def _mega_body(x0q0, x0q1, x0q2, x0q3, src_idx, dst_idx,
               deg_out, u_out, zb0, zb1, zb2, zb3,
               sidx_blk, didx_blk, rows, ones_v, zero_v, degbuf, xbuf, w,
               sem, acc):
    c = lax.axis_index("c")
    s = lax.axis_index("s")
    base = s * RPT

    _fill_rows(ones_v, CHUNK, 1.0)
    _fill_rows(zero_v, BLK, 0.0)

    def zero_acc():
        def blk(b, carry):
            pltpu.sync_copy(zero_v, acc.at[pl.ds(base + b * BLK, BLK)])
            return carry

        lax.fori_loop(0, NBLK, blk, 0)

    zero_acc()
    plsc.subcore_barrier()

    def deg_blk(jb, carry):
        pltpu.sync_copy(dst_idx.at[s, pl.ds(jb * JBLK, JBLK)], didx_blk)

        def chunk(jj, carry2):
            pltpu.sync_copy(ones_v, acc.at[didx_blk.at[jj]], add=True)
            return carry2

        lax.fori_loop(0, JBLK, chunk, 0)
        return carry

    lax.fori_loop(0, NJB, deg_blk, 0)
    plsc.subcore_barrier()

    sl = pl.ds(base, RPT)

    @pl.when(c == 0)
    def _():
        pltpu.sync_copy(acc.at[sl], deg_out.at[sl])

    def scale_quarter(x0src, zdst, store_w):
        def blk(b, carry):
            r0 = b * BLK
            pltpu.sync_copy(acc.at[pl.ds(base + r0, BLK)], degbuf)
            pltpu.sync_copy(x0src.at[pl.ds(base + r0, BLK)], xbuf)

            def row(i, carry2):
                d = degbuf[i, :]
                dis = jnp.where(d > 0, _rsqrt16(jnp.maximum(d, 1.0)), 0.0)
                if store_w:
                    w[r0 + i, :] = dis * dis
                xbuf[i, :] = dis * xbuf[i, :]
                return carry2

            lax.fori_loop(0, BLK, row, 0)
            pltpu.sync_copy(xbuf, zdst.at[pl.ds(base + r0, BLK)])
            return carry

        lax.fori_loop(0, NBLK, blk, 0)

    @pl.when(c == 0)
    def _():
        scale_quarter(x0q0, zb0, True)
        scale_quarter(x0q2, zb2, False)

    @pl.when(c == 1)
    def _():
        scale_quarter(x0q1, zb1, True)
        scale_quarter(x0q3, zb3, False)

    def scatter_pass(zsrc):
        def jblk(jb, carry):
            j0 = jb * JBLK
            pltpu.sync_copy(src_idx.at[s, pl.ds(j0, JBLK)], sidx_blk)
            pltpu.sync_copy(dst_idx.at[s, pl.ds(j0, JBLK)], didx_blk)

            def chunk(jj, carry2):
                pltpu.async_copy(zsrc.at[sidx_blk.at[jj]], rows, sem).wait()
                pltpu.sync_copy(rows, acc.at[didx_blk.at[jj]], add=True)
                return carry2

            lax.fori_loop(0, JBLK, chunk, 0)
            return carry

        lax.fori_loop(0, NJB, jblk, 0)

    def copy_out(kq, zdst, rescale):
        def blk(b, carry):
            r0 = b * BLK
            pltpu.sync_copy(acc.at[pl.ds(base + r0, BLK)], xbuf)
            pltpu.sync_copy(xbuf, u_out.at[kq, pl.ds(base + r0, BLK)])
            if rescale:
                def row(i, carry2):
                    xbuf[i, :] = w[r0 + i, :] * xbuf[i, :]
                    return carry2

                lax.fori_loop(0, BLK, row, 0)
                pltpu.sync_copy(xbuf, zdst.at[pl.ds(base + r0, BLK)])
            return carry

        lax.fori_loop(0, NBLK, blk, 0)

    for k in range(N_LAYERS):
        rescale = k + 1 < N_LAYERS
        for p in range(2):

            @pl.when(c == 0)
            def _(p=p, k=k, rescale=rescale):
                zq = (zb0, zb2)[p]
                zero_acc()
                plsc.subcore_barrier()
                scatter_pass(zq)
                plsc.subcore_barrier()
                copy_out(k * 4 + 2 * p, zq, rescale)

            @pl.when(c == 1)
            def _(p=p, k=k, rescale=rescale):
                zq = (zb1, zb3)[p]
                zero_acc()
                plsc.subcore_barrier()
                scatter_pass(zq)
                plsc.subcore_barrier()
                copy_out(k * 4 + 2 * p + 1, zq, rescale)


def _run_mega(x0qs, srcP, dstP):
    q16 = jax.ShapeDtypeStruct((N_PAD, QW), jnp.float32)
    return pl.kernel(
        _mega_body,
        out_type=(
            q16,
            jax.ShapeDtypeStruct((N_LAYERS * 4, N_PAD, QW), jnp.float32),
            q16, q16, q16, q16,
        ),
        mesh=_mesh(),
        compiler_params=_SC_PARAMS,
        scratch_types=[
            pltpu.VMEM((JBLK, CHUNK), jnp.int32),
            pltpu.VMEM((JBLK, CHUNK), jnp.int32),
            pltpu.VMEM((CHUNK, QW), jnp.float32),
            pltpu.VMEM((CHUNK, QW), jnp.float32),
            pltpu.VMEM((BLK, QW), jnp.float32),
            pltpu.VMEM((BLK, QW), jnp.float32),
            pltpu.VMEM((BLK, QW), jnp.float32),
            pltpu.VMEM((RPT, QW), jnp.float32),
            pltpu.SemaphoreType.DMA,
            pltpu.VMEM_SHARED((N_PAD, QW), jnp.float32),
        ],
    )(*x0qs, srcP, dstP)


def _final_body(table, src_idx, dst_idx, outf,
                sidx_all, didx_all, bufs, bufd, sems, semd):
    c = lax.axis_index("c")
    s = lax.axis_index("s")
    wid = c * NS + s
    pltpu.sync_copy(src_idx.at[wid], sidx_all)
    pltpu.sync_copy(dst_idx.at[wid], didx_all)
    base = wid * FCH * CHUNK

    def chunk(j, carry):
        cs = pltpu.async_copy(table.at[sidx_all.at[j]], bufs, sems)
        cd = pltpu.async_copy(table.at[didx_all.at[j]], bufd, semd)
        cs.wait()
        cd.wait()
        row0 = base + j * CHUNK
        pltpu.sync_copy(bufs, outf.at[pl.ds(row0, CHUNK), pl.ds(0, N_EMB)])
        pltpu.sync_copy(bufd, outf.at[pl.ds(row0, CHUNK), pl.ds(N_EMB, N_EMB)])
        return carry

    lax.fori_loop(0, FCH, chunk, 0)


def _run_final(out_acc, srcF, dstF):
    return pl.kernel(
        _final_body,
        out_type=jax.ShapeDtypeStruct((F_PAD, 2 * N_EMB), jnp.float32),
        mesh=_mesh(),
        compiler_params=_SC_PARAMS,
        scratch_types=[
            pltpu.VMEM((FCH, CHUNK), jnp.int32),
            pltpu.VMEM((FCH, CHUNK), jnp.int32),
            pltpu.VMEM((CHUNK, N_EMB), jnp.float32),
            pltpu.VMEM((CHUNK, N_EMB), jnp.float32),
            pltpu.SemaphoreType.DMA,
            pltpu.SemaphoreType.DMA,
        ],
    )(out_acc, srcF, dstF)


def _prep_body(x_ref, wt_ref, b_ref, q0_ref, q1_ref, q2_ref, q3_ref, acc_ref):
    x = jnp.dot(x_ref[...], wt_ref[...],
                preferred_element_type=jnp.float32) + b_ref[...]
    acc_ref[...] = ALPHA * x
    for q, ref in enumerate((q0_ref, q1_ref, q2_ref, q3_ref)):
        ref[...] = x[:, q * QW:(q + 1) * QW]


def _q_specs():
    return [pl.BlockSpec((BN, QW), lambda i: (i, 0)) for _ in range(4)]


def _q_shapes():
    return [jax.ShapeDtypeStruct((N_PAD, QW), jnp.float32) for _ in range(4)]


def _run_prep(Xp, WT, b2):
    return pl.pallas_call(
        _prep_body,
        grid=(N_PAD // BN,),
        in_specs=[
            pl.BlockSpec((BN, N_FEAT), lambda i: (i, 0)),
            pl.BlockSpec((N_FEAT, N_EMB), lambda i: (0, 0)),
            pl.BlockSpec((1, N_EMB), lambda i: (0, 0)),
        ],
        out_specs=_q_specs() + [pl.BlockSpec((BN, N_EMB), lambda i: (i, 0))],
        out_shape=_q_shapes() + [
            jax.ShapeDtypeStruct((N_PAD, N_EMB), jnp.float32)],
    )(Xp, WT, b2)


def _post_body(acc0_ref, deg_ref, u_ref, out_ref):
    deg = deg_ref[:, 0:1]
    dis = jnp.where(deg > 0, lax.rsqrt(jnp.maximum(deg, 1e-12)), 0.0)
    cols = []
    for q in range(4):
        usum = u_ref[q] + u_ref[4 + q] + u_ref[8 + q]
        cols.append(ALPHA * dis * usum)
    out_ref[...] = acc0_ref[...] + jnp.concatenate(cols, axis=1)


def _run_post(acc0, deg, u_all):
    return pl.pallas_call(
        _post_body,
        grid=(N_PAD // BN,),
        in_specs=[
            pl.BlockSpec((BN, N_EMB), lambda i: (i, 0)),
            pl.BlockSpec((BN, QW), lambda i: (i, 0)),
            pl.BlockSpec((N_LAYERS * 4, BN, QW), lambda i: (0, i, 0)),
        ],
        out_specs=pl.BlockSpec((BN, N_EMB), lambda i: (i, 0)),
        out_shape=jax.ShapeDtypeStruct((N_PAD, N_EMB), jnp.float32),
    )(acc0, deg, u_all)


def _pad_to(a, n, val):
    return jnp.concatenate([a, jnp.full((n - a.shape[0],), val, a.dtype)])


def kernel(edge_index, X, W, b):
    src = edge_index[0]
    dst = edge_index[1]

    srcP = _pad_to(src, P_PAD, 0).reshape(NS, PCH, CHUNK)
    dstP = _pad_to(dst, P_PAD, N_NODES).reshape(NS, PCH, CHUNK)
    srcF = _pad_to(src, F_PAD, 0).reshape(NW, FCH, CHUNK)
    dstF = _pad_to(dst, F_PAD, 0).reshape(NW, FCH, CHUNK)
    Xp = jnp.pad(X, ((0, N_PAD - N_NODES), (0, 0)))
    WT = W.T
    b2 = b.reshape(1, N_EMB)

    *x0qs, acc0 = _run_prep(Xp, WT, b2)
    deg, u_all, _, _, _, _ = _run_mega(x0qs, srcP, dstP)
    out_acc = _run_post(acc0, deg, u_all)
    outf = _run_final(out_acc, srcF, dstF)
    return outf[:N_EDGES]


# speedup vs baseline: 7.5704x; 1.0475x over previous
import jax
import jax.numpy as jnp
from jax import lax
from jax.experimental import pallas as pl
from jax.experimental.pallas import tpu as pltpu
from jax.experimental.pallas import tpu_sc as plsc

N_NODES = 50000
N_EDGES = 800000
N_FEAT = 128
N_EMB = 64
N_LAYERS = 3
ALPHA = 1.0 / (N_LAYERS + 1)

NC = 2
NS = 16
NW = NC * NS
CHUNK = 128
QW = 16

N_PAD = 50176
RPT = N_PAD // NS
BLK = 196
NBLK = RPT // BLK

PCH = 392
P_PAD = NS * PCH * CHUNK
JBLK = 14
NJB = PCH // JBLK
FCH = 196
F_PAD = NW * FCH * CHUNK

BN = 1024


def _mesh():
    return plsc.VectorSubcoreMesh(core_axis_name="c", subcore_axis_name="s")


_SC_PARAMS = pltpu.CompilerParams(use_tc_tiling_on_sc=False)


def _fill_rows(ref, n, value):
    val = jnp.full((QW,), value, jnp.float32)

    def row(i, carry):
        ref[i, :] = val
        return carry

    lax.fori_loop(0, n, row, 0)


def _rsqrt16(d):
    i = lax.bitcast_convert_type(d, jnp.int32)
    i = 0x5F3759DF - lax.shift_right_logical(i, 1)
    y = lax.bitcast_convert_type(i, jnp.float32)
    for _ in range(3):
        y = y * (1.5 - 0.5 * d * y * y)
    return y


def _mega_body(x0q, src_idx, dst_idx,
               deg_out, u_out, zb, w2,
               sidx_b, didx_b, rows, ones_v, zero_v, degbuf, xbuf, wblk,
               sem, acc):
    c = lax.axis_index("c")
    s = lax.axis_index("s")
    base = s * RPT

    _fill_rows(ones_v, CHUNK, 1.0)
    _fill_rows(zero_v, BLK, 0.0)

    def zero_acc():
        def blk(b, carry):
            pltpu.sync_copy(zero_v, acc.at[pl.ds(base + b * BLK, BLK)])
            return carry

        lax.fori_loop(0, NBLK, blk, 0)

    zero_acc()
    plsc.subcore_barrier()

    def deg_blk(jb, carry):
        pltpu.sync_copy(dst_idx.at[s, pl.ds(jb * JBLK, JBLK)], didx_b)

        def chunk(i, carry2):
            pltpu.sync_copy(ones_v, acc.at[didx_b.at[i]], add=True)
            return carry2

        lax.fori_loop(0, JBLK, chunk, 0)
        return carry

    lax.fori_loop(0, NJB, deg_blk, 0)
    plsc.subcore_barrier()

    sl = pl.ds(base, RPT)

    @pl.when(c == 0)
    def _():
        pltpu.sync_copy(acc.at[sl], deg_out.at[sl])

    def scale_p(p, carry):
        q = c + 2 * p

        def blk(b, carry2):
            r0 = b * BLK
            pltpu.sync_copy(acc.at[pl.ds(base + r0, BLK)], degbuf)
            pltpu.sync_copy(x0q.at[q, pl.ds(base + r0, BLK)], xbuf)

            def row(i, carry3):
                d = degbuf[i, :]
                dis = jnp.where(d > 0, _rsqrt16(jnp.maximum(d, 1.0)), 0.0)
                wblk[i, :] = dis * dis
                xbuf[i, :] = dis * xbuf[i, :]
                return carry3

            lax.fori_loop(0, BLK, row, 0)

            @pl.when(p == 0)
            def _():
                pltpu.sync_copy(wblk, w2.at[c, pl.ds(base + r0, BLK)])

            pltpu.sync_copy(xbuf, zb.at[q, pl.ds(base + r0, BLK)])
            return carry2

        lax.fori_loop(0, NBLK, blk, 0)
        return carry

    lax.fori_loop(0, 2, scale_p, 0)

    def pass_m(m, carry):
        p = m % 2
        q = c + 2 * p
        kq = (m // 2) * 4 + 2 * p + c

        zero_acc()
        plsc.subcore_barrier()

        def gather(i, slot):
            return pltpu.async_copy(
                zb.at[q].at[sidx_b.at[i]],
                rows.at[pl.ds(slot * CHUNK, CHUNK)], sem)

        def scatter(i, slot):
            pltpu.sync_copy(rows.at[pl.ds(slot * CHUNK, CHUNK)],
                            acc.at[didx_b.at[i]], add=True)

        def jblk(jb, carry2):
            pltpu.sync_copy(src_idx.at[s, pl.ds(jb * JBLK, JBLK)], sidx_b)
            pltpu.sync_copy(dst_idx.at[s, pl.ds(jb * JBLK, JBLK)], didx_b)
            gather(0, 0).wait()

            def chunk(i, carry3):
                par = i % 2

                @pl.when(i + 1 < JBLK)
                def _():
                    d = gather(i + 1, 1 - par)
                    scatter(i, par)
                    d.wait()

                @pl.when(i + 1 == JBLK)
                def _():
                    scatter(i, par)

                return carry3

            lax.fori_loop(0, JBLK, chunk, 0)
            return carry2

        lax.fori_loop(0, NJB, jblk, 0)
        plsc.subcore_barrier()

        def blk(b, carry2):
            r0 = b * BLK
            pltpu.sync_copy(acc.at[pl.ds(base + r0, BLK)], xbuf)
            pltpu.sync_copy(xbuf, u_out.at[kq, pl.ds(base + r0, BLK)])

            @pl.when(m < 2 * (N_LAYERS - 1))
            def _():
                pltpu.sync_copy(w2.at[c, pl.ds(base + r0, BLK)], wblk)

                def row(i, carry3):
                    xbuf[i, :] = wblk[i, :] * xbuf[i, :]
                    return carry3

                lax.fori_loop(0, BLK, row, 0)
                pltpu.sync_copy(xbuf, zb.at[q, pl.ds(base + r0, BLK)])

            return carry2

        lax.fori_loop(0, NBLK, blk, 0)
        return carry

    lax.fori_loop(0, 2 * N_LAYERS, pass_m, 0)


def _run_mega(x0q, srcP, dstP):
    return pl.kernel(
        _mega_body,
        out_type=(
            jax.ShapeDtypeStruct((N_PAD, QW), jnp.float32),
            jax.ShapeDtypeStruct((N_LAYERS * 4, N_PAD, QW), jnp.float32),
            jax.ShapeDtypeStruct((4, N_PAD, QW), jnp.float32),
            jax.ShapeDtypeStruct((NC, N_PAD, QW), jnp.float32),
        ),
        mesh=_mesh(),
        compiler_params=_SC_PARAMS,
        scratch_types=[
            pltpu.VMEM((JBLK, CHUNK), jnp.int32),
            pltpu.VMEM((JBLK, CHUNK), jnp.int32),
            pltpu.VMEM((2 * CHUNK, QW), jnp.float32),
            pltpu.VMEM((CHUNK, QW), jnp.float32),
            pltpu.VMEM((BLK, QW), jnp.float32),
            pltpu.VMEM((BLK, QW), jnp.float32),
            pltpu.VMEM((BLK, QW), jnp.float32),
            pltpu.VMEM((BLK, QW), jnp.float32),
            pltpu.SemaphoreType.DMA,
            pltpu.VMEM_SHARED((N_PAD, QW), jnp.float32),
        ],
    )(x0q, srcP, dstP)


def _final_body(table, src_idx, dst_idx, outf,
                sidx_all, didx_all, bufs, bufd, sems, semd):
    c = lax.axis_index("c")
    s = lax.axis_index("s")
    wid = c * NS + s
    pltpu.sync_copy(src_idx.at[wid], sidx_all)
    pltpu.sync_copy(dst_idx.at[wid], didx_all)
    base = wid * FCH * CHUNK

    def chunk(j, carry):
        cs = pltpu.async_copy(table.at[sidx_all.at[j]], bufs, sems)
        cd = pltpu.async_copy(table.at[didx_all.at[j]], bufd, semd)
        cs.wait()
        cd.wait()
        row0 = base + j * CHUNK
        pltpu.sync_copy(bufs, outf.at[pl.ds(row0, CHUNK), pl.ds(0, N_EMB)])
        pltpu.sync_copy(bufd, outf.at[pl.ds(row0, CHUNK), pl.ds(N_EMB, N_EMB)])
        return carry

    lax.fori_loop(0, FCH, chunk, 0)


def _run_final(out_acc, srcF, dstF):
    return pl.kernel(
        _final_body,
        out_type=jax.ShapeDtypeStruct((F_PAD, 2 * N_EMB), jnp.float32),
        mesh=_mesh(),
        compiler_params=_SC_PARAMS,
        scratch_types=[
            pltpu.VMEM((FCH, CHUNK), jnp.int32),
            pltpu.VMEM((FCH, CHUNK), jnp.int32),
            pltpu.VMEM((CHUNK, N_EMB), jnp.float32),
            pltpu.VMEM((CHUNK, N_EMB), jnp.float32),
            pltpu.SemaphoreType.DMA,
            pltpu.SemaphoreType.DMA,
        ],
    )(out_acc, srcF, dstF)


def _prep_body(x_ref, wt_ref, b_ref, q_ref, acc_ref):
    x = jnp.dot(x_ref[...], wt_ref[...],
                preferred_element_type=jnp.float32) + b_ref[...]
    acc_ref[...] = ALPHA * x
    for q in range(4):
        q_ref[q] = x[:, q * QW:(q + 1) * QW]


def _run_prep(Xp, WT, b2):
    return pl.pallas_call(
        _prep_body,
        grid=(N_PAD // BN,),
        in_specs=[
            pl.BlockSpec((BN, N_FEAT), lambda i: (i, 0)),
            pl.BlockSpec((N_FEAT, N_EMB), lambda i: (0, 0)),
            pl.BlockSpec((1, N_EMB), lambda i: (0, 0)),
        ],
        out_specs=[
            pl.BlockSpec((4, BN, QW), lambda i: (0, i, 0)),
            pl.BlockSpec((BN, N_EMB), lambda i: (i, 0)),
        ],
        out_shape=[
            jax.ShapeDtypeStruct((4, N_PAD, QW), jnp.float32),
            jax.ShapeDtypeStruct((N_PAD, N_EMB), jnp.float32),
        ],
    )(Xp, WT, b2)


def _post_body(acc0_ref, deg_ref, u_ref, out_ref):
    deg = deg_ref[:, 0:1]
    dis = jnp.where(deg > 0, lax.rsqrt(jnp.maximum(deg, 1e-12)), 0.0)
    cols = []
    for q in range(4):
        usum = u_ref[q] + u_ref[4 + q] + u_ref[8 + q]
        cols.append(ALPHA * dis * usum)
    out_ref[...] = acc0_ref[...] + jnp.concatenate(cols, axis=1)


def _run_post(acc0, deg, u_all):
    return pl.pallas_call(
        _post_body,
        grid=(N_PAD // BN,),
        in_specs=[
            pl.BlockSpec((BN, N_EMB), lambda i: (i, 0)),
            pl.BlockSpec((BN, QW), lambda i: (i, 0)),
            pl.BlockSpec((N_LAYERS * 4, BN, QW), lambda i: (0, i, 0)),
        ],
        out_specs=pl.BlockSpec((BN, N_EMB), lambda i: (i, 0)),
        out_shape=jax.ShapeDtypeStruct((N_PAD, N_EMB), jnp.float32),
    )(acc0, deg, u_all)


def _pad_to(a, n, val):
    return jnp.concatenate([a, jnp.full((n - a.shape[0],), val, a.dtype)])


def kernel(edge_index, X, W, b):
    src = edge_index[0]
    dst = edge_index[1]

    srcP = _pad_to(src, P_PAD, 0).reshape(NS, PCH, CHUNK)
    dstP = _pad_to(dst, P_PAD, N_NODES).reshape(NS, PCH, CHUNK)
    srcF = _pad_to(src, F_PAD, 0).reshape(NW, FCH, CHUNK)
    dstF = _pad_to(dst, F_PAD, 0).reshape(NW, FCH, CHUNK)
    Xp = jnp.pad(X, ((0, N_PAD - N_NODES), (0, 0)))
    WT = W.T
    b2 = b.reshape(1, N_EMB)

    x0q, acc0 = _run_prep(Xp, WT, b2)
    deg, u_all = _run_mega(x0q, srcP, dstP)[:2]
    out_acc = _run_post(acc0, deg, u_all)
    outf = _run_final(out_acc, srcF, dstF)
    return outf[:N_EDGES]
